# baseline (speedup 1.0000x reference)
import functools

import jax
import jax.numpy as jnp
from jax import lax
from jax.experimental import pallas as pl
from jax.experimental.pallas import tpu as pltpu
from jax.experimental.pallas import tpu_sc as plsc

N = 10000
E = 320000
D = 128
H = 4
C = 64
HC = H * C
HID = 128

NC = 2
NS = 16
LANES = 16

NPAD = 10240
W_ROW = 144
EK = 64
E2 = 331776
PAD_NODE = 10016

EPT = E2 // NS
NCH = EPT // EK
RPT = NPAD // NS
JN = NCH // 4

ROWBLK = 2000
NBLK = N // ROWBLK


PB = 2048
NPB = NPAD // PB


def _tc_prep(h_ref, w0_ref, w1_ref, asd0_ref, asd1_ref,
             xp00, xp01, xp10, xp11, alp0, alp1):
    hv = h_ref[...]
    z2 = jnp.zeros((PB, 2), jnp.float32)
    z12 = jnp.zeros((PB, W_ROW - D - 4), jnp.float32)
    for w_ref, asd_ref, xa, xb, alp in (
        (w0_ref, asd0_ref, xp00, xp01, alp0),
        (w1_ref, asd1_ref, xp10, xp11, alp1),
    ):
        xp = lax.dot_general(hv, w_ref[...], (((1,), (0,)), ((), ())),
                             precision=lax.Precision.HIGHEST,
                             preferred_element_type=jnp.float32)
        al = lax.dot_general(xp, asd_ref[...], (((1,), (0,)), ((), ())),
                             precision=lax.Precision.HIGHEST,
                             preferred_element_type=jnp.float32)
        alp[...] = al
        xa[:, pl.ds(0, D)] = xp[:, :D]
        xa[:, pl.ds(D, W_ROW - D)] = jnp.concatenate(
            [z2, al[:, 0:2], z12], axis=1)
        xb[:, pl.ds(0, D)] = xp[:, D:]
        xb[:, pl.ds(D, W_ROW - D)] = jnp.concatenate(
            [z2, al[:, 2:4], z12], axis=1)


_tc_prep_call = pl.pallas_call(
    _tc_prep,
    grid=(NPB,),
    in_specs=[
        pl.BlockSpec((PB, D), lambda i: (i, 0)),
        pl.BlockSpec((D, HC), lambda i: (0, 0)),
        pl.BlockSpec((D, HC), lambda i: (0, 0)),
        pl.BlockSpec((HC, 16), lambda i: (0, 0)),
        pl.BlockSpec((HC, 16), lambda i: (0, 0)),
    ],
    out_specs=[
        pl.BlockSpec((PB, W_ROW), lambda i: (i, 0)),
        pl.BlockSpec((PB, W_ROW), lambda i: (i, 0)),
        pl.BlockSpec((PB, W_ROW), lambda i: (i, 0)),
        pl.BlockSpec((PB, W_ROW), lambda i: (i, 0)),
        pl.BlockSpec((PB, 16), lambda i: (i, 0)),
        pl.BlockSpec((PB, 16), lambda i: (i, 0)),
    ],
    out_shape=(
        jax.ShapeDtypeStruct((NPAD, W_ROW), jnp.float32),
        jax.ShapeDtypeStruct((NPAD, W_ROW), jnp.float32),
        jax.ShapeDtypeStruct((NPAD, W_ROW), jnp.float32),
        jax.ShapeDtypeStruct((NPAD, W_ROW), jnp.float32),
        jax.ShapeDtypeStruct((NPAD, 16), jnp.float32),
        jax.ShapeDtypeStruct((NPAD, 16), jnp.float32),
    ),
)


def _sc_gat(idx0_hbm, idx1_hbm, al0_hbm, al1_hbm,
            xp00_hbm, xp01_hbm, xp10_hbm, xp11_hbm, out_hbm,
            idx0_v, idx1_v, idx2_v, idx3_v,
            ald0_v, rows0_v, ald1_v, rows1_v,
            acc_sp,
            isem0, isem1, isem2, isem3, gsem0, gsem1, ssem0, ssem1):
    c = lax.axis_index("c")
    s = lax.axis_index("s")
    idxs = (idx0_v, idx1_v, idx2_v, idx3_v)
    isems = (isem0, isem1, isem2, isem3)
    rbufs = ((ald0_v, rows0_v, gsem0, ssem0),
             (ald1_v, rows1_v, gsem1, ssem1))
    mps = ((idx0_hbm, al0_hbm, xp00_hbm, xp01_hbm),
           (idx1_hbm, al1_hbm, xp10_hbm, xp11_hbm))

    r0 = s * RPT
    cbase = s * NCH
    hh0 = 2 * c
    lane_iota = lax.iota(jnp.int32, LANES)

    for mp in range(2):
        idx_hbm, al_hbm, xp0_hbm, xp1_hbm = mps[mp]
        _sc_gat_one(idx_hbm, al_hbm, xp0_hbm, xp1_hbm, out_hbm,
                    idxs, isems, rbufs, acc_sp,
                    mp, c, r0, cbase, hh0, lane_iota)


def _sc_gat_one(idx_hbm, al_hbm, xp0_hbm, xp1_hbm, out_hbm,
                idxs, isems, rbufs, acc_sp,
                mp, c, r0, cbase, hh0, lane_iota):
    rows0_v = rbufs[0][1]

    def zrow(r, carry):
        for q in range(W_ROW // LANES):
            rows0_v[r, pl.ds(q * LANES, LANES)] = jnp.zeros((LANES,),
                                                            jnp.float32)
        return carry
    lax.fori_loop(0, EK, zrow, 0)
    for blk in range(RPT // EK):
        pltpu.sync_copy(rows0_v, acc_sp.at[pl.ds(r0 + blk * EK, EK)])
    plsc.subcore_barrier()

    def idx_load(k, slot):
        pltpu.async_copy(idx_hbm.at[cbase + k], idxs[slot], isems[slot])

    def idx_wait(slot):
        pltpu.make_async_copy(idx_hbm.at[cbase], idxs[slot],
                              isems[slot]).wait()

    def start_gathers(slot, ald_v, rows_v, gsem):
        iv = idxs[slot]

        @pl.when(c == 0)
        def _():
            pltpu.async_copy(xp0_hbm.at[iv.at[0]], rows_v, gsem)

        @pl.when(c == 1)
        def _():
            pltpu.async_copy(xp1_hbm.at[iv.at[0]], rows_v, gsem)

        pltpu.async_copy(al_hbm.at[iv.at[1]], ald_v, gsem)

    def wait_gathers(slot, ald_v, rows_v, gsem):
        iv = idxs[slot]
        pltpu.make_async_copy(xp0_hbm.at[iv.at[0]], rows_v, gsem).wait()
        pltpu.make_async_copy(al_hbm.at[iv.at[1]], ald_v, gsem).wait()

    def scat(slot, rows_v, ssem):
        pltpu.async_copy(rows_v, acc_sp.at[idxs[slot].at[1]], ssem, add=True)

    def scat_wait(slot, rows_v, ssem):
        pltpu.make_async_copy(rows_v, acc_sp.at[idxs[slot].at[1]],
                              ssem).wait()

    def compute_chunk(ald_v, rows_v):
        for g in range(EK // LANES):
            erow = lane_iota + g * LANES
            for h in range(2):
                a_s = plsc.load_gather(
                    rows_v, [erow, jnp.full((LANES,), D + 2 + h, jnp.int32)])
                a_d = plsc.load_gather(
                    ald_v,
                    [erow, jnp.full((LANES,), H + h, jnp.int32) + hh0])
                al = a_s + a_d
                alpha = jnp.where(al >= 0.0, al, 0.2 * al)
                ex = jnp.exp(alpha)
                plsc.store_scatter(
                    rows_v, [erow, jnp.full((LANES,), D + h, jnp.int32)], ex)

        for e in range(EK):
            exv = rows_v[e, pl.ds(D, LANES)]
            b0 = jnp.full((LANES,), exv[0], jnp.float32)
            b1 = jnp.full((LANES,), exv[1], jnp.float32)
            for q in range(C // LANES):
                rows_v[e, pl.ds(q * LANES, LANES)] = (
                    rows_v[e, pl.ds(q * LANES, LANES)] * b0)
            for q in range(C // LANES, 2 * C // LANES):
                rows_v[e, pl.ds(q * LANES, LANES)] = (
                    rows_v[e, pl.ds(q * LANES, LANES)] * b1)

    idx_load(0, 0)
    idx_load(1, 1)
    idx_load(2, 2)
    idx_wait(0)
    start_gathers(0, rbufs[0][0], rbufs[0][1], rbufs[0][2])

    def body(j, carry):
        for b in range(4):
            k = 4 * j + b
            A = rbufs[b % 2]
            B = rbufs[1 - (b % 2)]
            sl = b
            sln = (b + 1) % 4
            slf = (b + 3) % 4

            def drain_prev():
                scat_wait(slf, B[1], B[3])

            if b == 0:
                @pl.when(j > 0)
                def _():
                    drain_prev()
            else:
                drain_prev()

            if b == 0:
                idx_load(k + 3, slf)
            else:
                @pl.when(j < JN - 1)
                def _():
                    idx_load(k + 3, slf)

            def fetch_next():
                idx_wait(sln)
                start_gathers(sln, B[0], B[1], B[2])

            if b < 3:
                fetch_next()
            else:
                @pl.when(j < JN - 1)
                def _():
                    fetch_next()

            wait_gathers(sl, A[0], A[1], A[2])
            compute_chunk(A[0], A[1])
            scat(sl, A[1], A[3])
        return carry

    lax.fori_loop(0, JN, body, 0)

    scat_wait(3, rbufs[1][1], rbufs[1][3])
    plsc.subcore_barrier()

    pltpu.sync_copy(acc_sp.at[pl.ds(r0, RPT)],
                    out_hbm.at[mp, c, pl.ds(r0, RPT)])


@functools.lru_cache(maxsize=1)
def _sc_gat_call():
    return pl.kernel(
        _sc_gat,
        out_type=jax.ShapeDtypeStruct((2, NC, NPAD, W_ROW), jnp.float32),
        mesh=plsc.VectorSubcoreMesh(core_axis_name="c", subcore_axis_name="s",
                                    num_cores=NC, num_subcores=NS),
        compiler_params=pltpu.CompilerParams(needs_layout_passes=False,
                                             use_tc_tiling_on_sc=False),
        scratch_types=[
            pltpu.VMEM((2, EK), jnp.int32),
            pltpu.VMEM((2, EK), jnp.int32),
            pltpu.VMEM((2, EK), jnp.int32),
            pltpu.VMEM((2, EK), jnp.int32),
            pltpu.VMEM((EK, 16), jnp.float32),
            pltpu.VMEM((EK, W_ROW), jnp.float32),
            pltpu.VMEM((EK, 16), jnp.float32),
            pltpu.VMEM((EK, W_ROW), jnp.float32),
            pltpu.VMEM_SHARED((NPAD, W_ROW), jnp.float32),
            pltpu.SemaphoreType.DMA,
            pltpu.SemaphoreType.DMA,
            pltpu.SemaphoreType.DMA,
            pltpu.SemaphoreType.DMA,
            pltpu.SemaphoreType.DMA,
            pltpu.SemaphoreType.DMA,
            pltpu.SemaphoreType.DMA,
            pltpu.SemaphoreType.DMA,
        ],
    )


def _tc_post(acc0_ref, acc1_ref, b0_ref, b1_ref,
             saw1_ref, sab1_ref, saw2_ref, z0_ref, z1_ref, wsum_ref):
    i = pl.program_id(0)
    saw1 = saw1_ref[...]
    sab1 = sab1_ref[...]
    saw2 = saw2_ref[...]
    wparts = []
    for acc_ref, b_ref, z_ref in (
            (acc0_ref, b0_ref, z0_ref),
            (acc1_ref, b1_ref, z1_ref)):
        acc = acc_ref[...]
        segs = []
        for sc in range(2):
            for h in range(2):
                num = acc[sc, :, h * C:(h + 1) * C]
                den = acc[sc, :, D + h][:, None]
                segs.append(num / (den + 1e-16))
        z = jnp.concatenate(segs, axis=1) + b_ref[...]
        z_ref[...] = z
        t = jnp.tanh(
            lax.dot_general(z, saw1, (((1,), (0,)), ((), ())),
                            precision=lax.Precision.HIGHEST,
                            preferred_element_type=jnp.float32) + sab1)
        wparts.append(jnp.sum(t * saw2))
    wnew = jnp.stack([jnp.full((128,), wparts[0], jnp.float32),
                      jnp.full((128,), wparts[1], jnp.float32)])

    @pl.when(i == 0)
    def _():
        wsum_ref[...] = wnew

    @pl.when(i > 0)
    def _():
        wsum_ref[...] = wsum_ref[...] + wnew


_tc_post_call = pl.pallas_call(
    _tc_post,
    grid=(NBLK,),
    in_specs=[
        pl.BlockSpec((NC, ROWBLK, W_ROW), lambda i: (0, i, 0)),
        pl.BlockSpec((NC, ROWBLK, W_ROW), lambda i: (0, i, 0)),
        pl.BlockSpec((1, HC), lambda i: (0, 0)),
        pl.BlockSpec((1, HC), lambda i: (0, 0)),
        pl.BlockSpec((HC, HID), lambda i: (0, 0)),
        pl.BlockSpec((1, HID), lambda i: (0, 0)),
        pl.BlockSpec((1, HID), lambda i: (0, 0)),
    ],
    out_specs=[
        pl.BlockSpec((ROWBLK, HC), lambda i: (i, 0)),
        pl.BlockSpec((ROWBLK, HC), lambda i: (i, 0)),
        pl.BlockSpec((2, 128), lambda i: (0, 0)),
    ],
    out_shape=(
        jax.ShapeDtypeStruct((N, HC), jnp.float32),
        jax.ShapeDtypeStruct((N, HC), jnp.float32),
        jax.ShapeDtypeStruct((2, 128), jnp.float32),
    ),
)


def _tc_combine(z0_ref, z1_ref, wsum_ref, out_ref):
    w = wsum_ref[...]
    a0 = w[0:1, 0:1] / N
    a1 = w[1:2, 0:1] / N
    m = jnp.maximum(a0, a1)
    e0 = jnp.exp(a0 - m)
    e1 = jnp.exp(a1 - m)
    tot = e0 + e1
    out_ref[...] = (z0_ref[...] * (e0 / tot) + z1_ref[...] * (e1 / tot))


_tc_combine_call = pl.pallas_call(
    _tc_combine,
    grid=(NBLK,),
    in_specs=[
        pl.BlockSpec((ROWBLK, HC), lambda i: (i, 0)),
        pl.BlockSpec((ROWBLK, HC), lambda i: (i, 0)),
        pl.BlockSpec((2, 128), lambda i: (0, 0)),
    ],
    out_specs=pl.BlockSpec((ROWBLK, HC), lambda i: (i, 0)),
    out_shape=jax.ShapeDtypeStruct((N, HC), jnp.float32),
)


def kernel(h, edge_index_0, edge_index_1, W0, as0, ad0, b0,
           W1, as1, ad1, b1, saW1, sab1, saW2):
    rows = jnp.arange(HC)
    headid = (rows // C).astype(jnp.int32)
    asd0 = jnp.zeros((HC, 16), jnp.float32)
    asd0 = asd0.at[rows, headid].set(as0.reshape(HC))
    asd0 = asd0.at[rows, H + headid].set(ad0.reshape(HC))
    asd1 = jnp.zeros((HC, 16), jnp.float32)
    asd1 = asd1.at[rows, headid].set(as1.reshape(HC))
    asd1 = asd1.at[rows, H + headid].set(ad1.reshape(HC))

    loop = jnp.arange(N, dtype=jnp.int32)
    pad = jnp.full((E2 - E - N,), PAD_NODE, jnp.int32)

    def pack(ei):
        src = jnp.concatenate([ei[0].astype(jnp.int32), loop, pad])
        dst = jnp.concatenate([ei[1].astype(jnp.int32), loop, pad])
        return jnp.stack([src.reshape(-1, EK), dst.reshape(-1, EK)], axis=1)

    idx0 = pack(edge_index_0)
    idx1 = pack(edge_index_1)

    h_pad = jnp.zeros((NPAD, D), jnp.float32).at[:N].set(h)
    xp00, xp01, xp10, xp11, alp0, alp1 = _tc_prep_call(
        h_pad, W0, W1, asd0, asd1)

    sc_gat = _sc_gat_call()
    acc = sc_gat(idx0, idx1, alp0, alp1, xp00, xp01, xp10, xp11)
    acc0 = acc[0]
    acc1 = acc[1]

    z0, z1, wsum = _tc_post_call(
        acc0, acc1, b0.reshape(1, HC), b1.reshape(1, HC),
        saW1, sab1.reshape(1, HID), saW2.reshape(1, HID))

    return _tc_combine_call(z0, z1, wsum)


# speedup vs baseline: 1.0779x; 1.0779x over previous
import functools

import jax
import jax.numpy as jnp
from jax import lax
from jax.experimental import pallas as pl
from jax.experimental.pallas import tpu as pltpu
from jax.experimental.pallas import tpu_sc as plsc

N = 10000
E = 320000
D = 128
H = 4
C = 64
HC = H * C
HID = 128

NC = 2
NS = 16
LANES = 16

NPAD = 10240
W_ROW = 144
EK = 64
E2 = 331776
PAD_NODE = 10016

EPT = E2 // NS
NCH = EPT // EK
RPT = NPAD // NS
JN = NCH // 4

ROWBLK = 2000
NBLK = N // ROWBLK


PB = 2048
NPB = NPAD // PB


def _tc_prep(h_ref, w0_ref, w1_ref, asd0_ref, asd1_ref,
             xp00, xp01, xp10, xp11, alp0, alp1):
    hv = h_ref[...]
    z2 = jnp.zeros((PB, 2), jnp.float32)
    z12 = jnp.zeros((PB, W_ROW - D - 4), jnp.float32)
    for w_ref, asd_ref, xa, xb, alp in (
        (w0_ref, asd0_ref, xp00, xp01, alp0),
        (w1_ref, asd1_ref, xp10, xp11, alp1),
    ):
        xp = lax.dot_general(hv, w_ref[...], (((1,), (0,)), ((), ())),
                             precision=lax.Precision.HIGHEST,
                             preferred_element_type=jnp.float32)
        al = lax.dot_general(xp, asd_ref[...], (((1,), (0,)), ((), ())),
                             precision=lax.Precision.HIGHEST,
                             preferred_element_type=jnp.float32)
        alp[...] = al
        xa[:, pl.ds(0, D)] = xp[:, :D]
        xa[:, pl.ds(D, W_ROW - D)] = jnp.concatenate(
            [z2, al[:, 0:2], z12], axis=1)
        xb[:, pl.ds(0, D)] = xp[:, D:]
        xb[:, pl.ds(D, W_ROW - D)] = jnp.concatenate(
            [z2, al[:, 2:4], z12], axis=1)


_tc_prep_call = pl.pallas_call(
    _tc_prep,
    grid=(NPB,),
    in_specs=[
        pl.BlockSpec((PB, D), lambda i: (i, 0)),
        pl.BlockSpec((D, HC), lambda i: (0, 0)),
        pl.BlockSpec((D, HC), lambda i: (0, 0)),
        pl.BlockSpec((HC, 16), lambda i: (0, 0)),
        pl.BlockSpec((HC, 16), lambda i: (0, 0)),
    ],
    out_specs=[
        pl.BlockSpec((PB, W_ROW), lambda i: (i, 0)),
        pl.BlockSpec((PB, W_ROW), lambda i: (i, 0)),
        pl.BlockSpec((PB, W_ROW), lambda i: (i, 0)),
        pl.BlockSpec((PB, W_ROW), lambda i: (i, 0)),
        pl.BlockSpec((PB, 16), lambda i: (i, 0)),
        pl.BlockSpec((PB, 16), lambda i: (i, 0)),
    ],
    out_shape=(
        jax.ShapeDtypeStruct((NPAD, W_ROW), jnp.float32),
        jax.ShapeDtypeStruct((NPAD, W_ROW), jnp.float32),
        jax.ShapeDtypeStruct((NPAD, W_ROW), jnp.float32),
        jax.ShapeDtypeStruct((NPAD, W_ROW), jnp.float32),
        jax.ShapeDtypeStruct((NPAD, 16), jnp.float32),
        jax.ShapeDtypeStruct((NPAD, 16), jnp.float32),
    ),
)


def _sc_gat(idx_hbm, al_hbm, xp0_hbm, xp1_hbm, out_hbm,
            idx0_v, idx1_v, idx2_v, idx3_v,
            ald0_v, rows0_v, ald1_v, rows1_v,
            acc_sp,
            isem0, isem1, isem2, isem3, gsem0, gsem1, ssem0, ssem1):
    c = lax.axis_index("c")
    s = lax.axis_index("s")
    idxs = (idx0_v, idx1_v, idx2_v, idx3_v)
    isems = (isem0, isem1, isem2, isem3)
    rbufs = ((ald0_v, rows0_v, gsem0, ssem0),
             (ald1_v, rows1_v, gsem1, ssem1))

    r0 = s * RPT
    cbase = s * NCH
    hh0 = 2 * c
    lane_iota = lax.iota(jnp.int32, LANES)

    def zrow(r, carry):
        for q in range(W_ROW // LANES):
            rows0_v[r, pl.ds(q * LANES, LANES)] = jnp.zeros((LANES,),
                                                            jnp.float32)
        return carry
    lax.fori_loop(0, EK, zrow, 0)
    for blk in range(RPT // EK):
        pltpu.sync_copy(rows0_v, acc_sp.at[pl.ds(r0 + blk * EK, EK)])
    plsc.subcore_barrier()

    def idx_load(k, slot):
        pltpu.async_copy(idx_hbm.at[cbase + k], idxs[slot], isems[slot])

    def idx_wait(slot):
        pltpu.make_async_copy(idx_hbm.at[cbase], idxs[slot],
                              isems[slot]).wait()

    def start_gathers(slot, ald_v, rows_v, gsem):
        iv = idxs[slot]

        @pl.when(c == 0)
        def _():
            pltpu.async_copy(xp0_hbm.at[iv.at[0]], rows_v, gsem)

        @pl.when(c == 1)
        def _():
            pltpu.async_copy(xp1_hbm.at[iv.at[0]], rows_v, gsem)

        pltpu.async_copy(al_hbm.at[iv.at[1]], ald_v, gsem)

    def wait_gathers(slot, ald_v, rows_v, gsem):
        iv = idxs[slot]
        pltpu.make_async_copy(xp0_hbm.at[iv.at[0]], rows_v, gsem).wait()
        pltpu.make_async_copy(al_hbm.at[iv.at[1]], ald_v, gsem).wait()

    def scat(slot, rows_v, ssem):
        pltpu.async_copy(rows_v, acc_sp.at[idxs[slot].at[1]], ssem, add=True)

    def scat_wait(slot, rows_v, ssem):
        pltpu.make_async_copy(rows_v, acc_sp.at[idxs[slot].at[1]],
                              ssem).wait()

    def compute_chunk(ald_v, rows_v):
        for g in range(EK // LANES):
            erow = lane_iota + g * LANES
            for h in range(2):
                a_s = plsc.load_gather(
                    rows_v, [erow, jnp.full((LANES,), D + 2 + h, jnp.int32)])
                a_d = plsc.load_gather(
                    ald_v,
                    [erow, jnp.full((LANES,), H + h, jnp.int32) + hh0])
                al = a_s + a_d
                alpha = jnp.where(al >= 0.0, al, 0.2 * al)
                ex = jnp.exp(alpha)
                plsc.store_scatter(
                    rows_v, [erow, jnp.full((LANES,), D + h, jnp.int32)], ex)

        for e in range(EK):
            exv = rows_v[e, pl.ds(D, LANES)]
            b0 = jnp.full((LANES,), exv[0], jnp.float32)
            b1 = jnp.full((LANES,), exv[1], jnp.float32)
            for q in range(C // LANES):
                rows_v[e, pl.ds(q * LANES, LANES)] = (
                    rows_v[e, pl.ds(q * LANES, LANES)] * b0)
            for q in range(C // LANES, 2 * C // LANES):
                rows_v[e, pl.ds(q * LANES, LANES)] = (
                    rows_v[e, pl.ds(q * LANES, LANES)] * b1)

    idx_load(0, 0)
    idx_load(1, 1)
    idx_load(2, 2)
    idx_wait(0)
    start_gathers(0, rbufs[0][0], rbufs[0][1], rbufs[0][2])

    def body(j, carry):
        for b in range(4):
            k = 4 * j + b
            A = rbufs[b % 2]
            B = rbufs[1 - (b % 2)]
            sl = b
            sln = (b + 1) % 4
            slf = (b + 3) % 4

            def drain_prev():
                scat_wait(slf, B[1], B[3])

            if b == 0:
                @pl.when(j > 0)
                def _():
                    drain_prev()
            else:
                drain_prev()

            if b == 0:
                idx_load(k + 3, slf)
            else:
                @pl.when(j < JN - 1)
                def _():
                    idx_load(k + 3, slf)

            def fetch_next():
                idx_wait(sln)
                start_gathers(sln, B[0], B[1], B[2])

            if b < 3:
                fetch_next()
            else:
                @pl.when(j < JN - 1)
                def _():
                    fetch_next()

            wait_gathers(sl, A[0], A[1], A[2])
            compute_chunk(A[0], A[1])
            scat(sl, A[1], A[3])
        return carry

    lax.fori_loop(0, JN, body, 0)

    scat_wait(3, rbufs[1][1], rbufs[1][3])
    plsc.subcore_barrier()

    pltpu.sync_copy(acc_sp.at[pl.ds(r0, RPT)],
                    out_hbm.at[c, pl.ds(r0, RPT)])


@functools.lru_cache(maxsize=1)
def _sc_gat_call():
    return pl.kernel(
        _sc_gat,
        out_type=jax.ShapeDtypeStruct((NC, NPAD, W_ROW), jnp.float32),
        mesh=plsc.VectorSubcoreMesh(core_axis_name="c", subcore_axis_name="s",
                                    num_cores=NC, num_subcores=NS),
        compiler_params=pltpu.CompilerParams(needs_layout_passes=False,
                                             use_tc_tiling_on_sc=False),
        scratch_types=[
            pltpu.VMEM((2, EK), jnp.int32),
            pltpu.VMEM((2, EK), jnp.int32),
            pltpu.VMEM((2, EK), jnp.int32),
            pltpu.VMEM((2, EK), jnp.int32),
            pltpu.VMEM((EK, 16), jnp.float32),
            pltpu.VMEM((EK, W_ROW), jnp.float32),
            pltpu.VMEM((EK, 16), jnp.float32),
            pltpu.VMEM((EK, W_ROW), jnp.float32),
            pltpu.VMEM_SHARED((NPAD, W_ROW), jnp.float32),
            pltpu.SemaphoreType.DMA,
            pltpu.SemaphoreType.DMA,
            pltpu.SemaphoreType.DMA,
            pltpu.SemaphoreType.DMA,
            pltpu.SemaphoreType.DMA,
            pltpu.SemaphoreType.DMA,
            pltpu.SemaphoreType.DMA,
            pltpu.SemaphoreType.DMA,
        ],
    )


def _tc_post(acc0_ref, acc1_ref, b0_ref, b1_ref,
             saw1_ref, sab1_ref, saw2_ref, z0_ref, z1_ref, wsum_ref):
    i = pl.program_id(0)
    saw1 = saw1_ref[...]
    sab1 = sab1_ref[...]
    saw2 = saw2_ref[...]
    wparts = []
    for acc_ref, b_ref, z_ref in (
            (acc0_ref, b0_ref, z0_ref),
            (acc1_ref, b1_ref, z1_ref)):
        acc = acc_ref[...]
        segs = []
        for sc in range(2):
            for h in range(2):
                num = acc[sc, :, h * C:(h + 1) * C]
                den = acc[sc, :, D + h][:, None]
                segs.append(num / (den + 1e-16))
        z = jnp.concatenate(segs, axis=1) + b_ref[...]
        z_ref[...] = z
        t = jnp.tanh(
            lax.dot_general(z, saw1, (((1,), (0,)), ((), ())),
                            precision=lax.Precision.HIGHEST,
                            preferred_element_type=jnp.float32) + sab1)
        wparts.append(jnp.sum(t * saw2))
    wnew = jnp.stack([jnp.full((128,), wparts[0], jnp.float32),
                      jnp.full((128,), wparts[1], jnp.float32)])

    @pl.when(i == 0)
    def _():
        wsum_ref[...] = wnew

    @pl.when(i > 0)
    def _():
        wsum_ref[...] = wsum_ref[...] + wnew


_tc_post_call = pl.pallas_call(
    _tc_post,
    grid=(NBLK,),
    in_specs=[
        pl.BlockSpec((NC, ROWBLK, W_ROW), lambda i: (0, i, 0)),
        pl.BlockSpec((NC, ROWBLK, W_ROW), lambda i: (0, i, 0)),
        pl.BlockSpec((1, HC), lambda i: (0, 0)),
        pl.BlockSpec((1, HC), lambda i: (0, 0)),
        pl.BlockSpec((HC, HID), lambda i: (0, 0)),
        pl.BlockSpec((1, HID), lambda i: (0, 0)),
        pl.BlockSpec((1, HID), lambda i: (0, 0)),
    ],
    out_specs=[
        pl.BlockSpec((ROWBLK, HC), lambda i: (i, 0)),
        pl.BlockSpec((ROWBLK, HC), lambda i: (i, 0)),
        pl.BlockSpec((2, 128), lambda i: (0, 0)),
    ],
    out_shape=(
        jax.ShapeDtypeStruct((N, HC), jnp.float32),
        jax.ShapeDtypeStruct((N, HC), jnp.float32),
        jax.ShapeDtypeStruct((2, 128), jnp.float32),
    ),
)


def _tc_combine(z0_ref, z1_ref, wsum_ref, out_ref):
    w = wsum_ref[...]
    a0 = w[0:1, 0:1] / N
    a1 = w[1:2, 0:1] / N
    m = jnp.maximum(a0, a1)
    e0 = jnp.exp(a0 - m)
    e1 = jnp.exp(a1 - m)
    tot = e0 + e1
    out_ref[...] = (z0_ref[...] * (e0 / tot) + z1_ref[...] * (e1 / tot))


_tc_combine_call = pl.pallas_call(
    _tc_combine,
    grid=(NBLK,),
    in_specs=[
        pl.BlockSpec((ROWBLK, HC), lambda i: (i, 0)),
        pl.BlockSpec((ROWBLK, HC), lambda i: (i, 0)),
        pl.BlockSpec((2, 128), lambda i: (0, 0)),
    ],
    out_specs=pl.BlockSpec((ROWBLK, HC), lambda i: (i, 0)),
    out_shape=jax.ShapeDtypeStruct((N, HC), jnp.float32),
)


def kernel(h, edge_index_0, edge_index_1, W0, as0, ad0, b0,
           W1, as1, ad1, b1, saW1, sab1, saW2):
    rows = jnp.arange(HC)
    headid = (rows // C).astype(jnp.int32)
    asd0 = jnp.zeros((HC, 16), jnp.float32)
    asd0 = asd0.at[rows, headid].set(as0.reshape(HC))
    asd0 = asd0.at[rows, H + headid].set(ad0.reshape(HC))
    asd1 = jnp.zeros((HC, 16), jnp.float32)
    asd1 = asd1.at[rows, headid].set(as1.reshape(HC))
    asd1 = asd1.at[rows, H + headid].set(ad1.reshape(HC))

    loop = jnp.arange(N, dtype=jnp.int32)
    pad = jnp.full((E2 - E - N,), PAD_NODE, jnp.int32)

    def pack(ei):
        src = jnp.concatenate([ei[0].astype(jnp.int32), loop, pad])
        dst = jnp.concatenate([ei[1].astype(jnp.int32), loop, pad])
        return jnp.stack([src.reshape(-1, EK), dst.reshape(-1, EK)], axis=1)

    idx0 = pack(edge_index_0)
    idx1 = pack(edge_index_1)

    h_pad = jnp.zeros((NPAD, D), jnp.float32).at[:N].set(h)
    xp00, xp01, xp10, xp11, alp0, alp1 = _tc_prep_call(
        h_pad, W0, W1, asd0, asd1)

    sc_gat = _sc_gat_call()
    acc0 = sc_gat(idx0, alp0, xp00, xp01)
    acc1 = sc_gat(idx1, alp1, xp10, xp11)

    z0, z1, wsum = _tc_post_call(
        acc0, acc1, b0.reshape(1, HC), b1.reshape(1, HC),
        saW1, sab1.reshape(1, HID), saW2.reshape(1, HID))

    return _tc_combine_call(z0, z1, wsum)


# speedup vs baseline: 1.1735x; 1.0886x over previous
import functools

import jax
import jax.numpy as jnp
from jax import lax
from jax.experimental import pallas as pl
from jax.experimental.pallas import tpu as pltpu
from jax.experimental.pallas import tpu_sc as plsc

N = 10000
E = 320000
D = 128
H = 4
C = 64
HC = H * C
HID = 128

NC = 2
NS = 16
LANES = 16

NPAD = 10240
W_ROW = 144
EK = 64
E2 = 331776
PAD_NODE = 10016

EPT = E2 // NS
NCH = EPT // EK
RPT = NPAD // NS
JN = NCH // 6

ROWBLK = 2000
NBLK = N // ROWBLK


PB = 2048
NPB = NPAD // PB


def _tc_prep(h_ref, w0_ref, w1_ref, asd0_ref, asd1_ref,
             xp00, xp01, xp10, xp11, alp0, alp1):
    hv = h_ref[...]
    z2 = jnp.zeros((PB, 2), jnp.float32)
    z12 = jnp.zeros((PB, W_ROW - D - 4), jnp.float32)
    for w_ref, asd_ref, xa, xb, alp in (
        (w0_ref, asd0_ref, xp00, xp01, alp0),
        (w1_ref, asd1_ref, xp10, xp11, alp1),
    ):
        xp = lax.dot_general(hv, w_ref[...], (((1,), (0,)), ((), ())),
                             precision=lax.Precision.HIGHEST,
                             preferred_element_type=jnp.float32)
        al = lax.dot_general(xp, asd_ref[...], (((1,), (0,)), ((), ())),
                             precision=lax.Precision.HIGHEST,
                             preferred_element_type=jnp.float32)
        alp[...] = al
        xa[:, pl.ds(0, D)] = xp[:, :D]
        xa[:, pl.ds(D, W_ROW - D)] = jnp.concatenate(
            [z2, al[:, 0:2], z12], axis=1)
        xb[:, pl.ds(0, D)] = xp[:, D:]
        xb[:, pl.ds(D, W_ROW - D)] = jnp.concatenate(
            [z2, al[:, 2:4], z12], axis=1)


_tc_prep_call = pl.pallas_call(
    _tc_prep,
    grid=(NPB,),
    in_specs=[
        pl.BlockSpec((PB, D), lambda i: (i, 0)),
        pl.BlockSpec((D, HC), lambda i: (0, 0)),
        pl.BlockSpec((D, HC), lambda i: (0, 0)),
        pl.BlockSpec((HC, 16), lambda i: (0, 0)),
        pl.BlockSpec((HC, 16), lambda i: (0, 0)),
    ],
    out_specs=[
        pl.BlockSpec((PB, W_ROW), lambda i: (i, 0)),
        pl.BlockSpec((PB, W_ROW), lambda i: (i, 0)),
        pl.BlockSpec((PB, W_ROW), lambda i: (i, 0)),
        pl.BlockSpec((PB, W_ROW), lambda i: (i, 0)),
        pl.BlockSpec((PB, 16), lambda i: (i, 0)),
        pl.BlockSpec((PB, 16), lambda i: (i, 0)),
    ],
    out_shape=(
        jax.ShapeDtypeStruct((NPAD, W_ROW), jnp.float32),
        jax.ShapeDtypeStruct((NPAD, W_ROW), jnp.float32),
        jax.ShapeDtypeStruct((NPAD, W_ROW), jnp.float32),
        jax.ShapeDtypeStruct((NPAD, W_ROW), jnp.float32),
        jax.ShapeDtypeStruct((NPAD, 16), jnp.float32),
        jax.ShapeDtypeStruct((NPAD, 16), jnp.float32),
    ),
)


def _sc_gat(idx_hbm, al_hbm, xp0_hbm, xp1_hbm, out_hbm,
            idx0_v, idx1_v, idx2_v, idx3_v, idx4_v, idx5_v,
            ald0_v, rows0_v, ald1_v, rows1_v, ald2_v, rows2_v,
            acc_sp,
            isem0, isem1, isem2, isem3, isem4, isem5,
            gsem0, gsem1, gsem2, ssem0, ssem1, ssem2):
    c = lax.axis_index("c")
    s = lax.axis_index("s")
    idxs = (idx0_v, idx1_v, idx2_v, idx3_v, idx4_v, idx5_v)
    isems = (isem0, isem1, isem2, isem3, isem4, isem5)
    rbufs = ((ald0_v, rows0_v, gsem0, ssem0),
             (ald1_v, rows1_v, gsem1, ssem1),
             (ald2_v, rows2_v, gsem2, ssem2))

    r0 = s * RPT
    cbase = s * NCH
    hh0 = 2 * c
    lane_iota = lax.iota(jnp.int32, LANES)

    def zrow(r, carry):
        for q in range(W_ROW // LANES):
            rows0_v[r, pl.ds(q * LANES, LANES)] = jnp.zeros((LANES,),
                                                            jnp.float32)
        return carry
    lax.fori_loop(0, EK, zrow, 0)
    for blk in range(RPT // EK):
        pltpu.sync_copy(rows0_v, acc_sp.at[pl.ds(r0 + blk * EK, EK)])
    plsc.subcore_barrier()

    def idx_load(k, slot):
        pltpu.async_copy(idx_hbm.at[cbase + k], idxs[slot], isems[slot])

    def idx_wait(slot):
        pltpu.make_async_copy(idx_hbm.at[cbase], idxs[slot],
                              isems[slot]).wait()

    def start_gathers(slot, ald_v, rows_v, gsem):
        iv = idxs[slot]

        @pl.when(c == 0)
        def _():
            pltpu.async_copy(xp0_hbm.at[iv.at[0]], rows_v, gsem)

        @pl.when(c == 1)
        def _():
            pltpu.async_copy(xp1_hbm.at[iv.at[0]], rows_v, gsem)

        pltpu.async_copy(al_hbm.at[iv.at[1]], ald_v, gsem)

    def wait_gathers(slot, ald_v, rows_v, gsem):
        iv = idxs[slot]
        pltpu.make_async_copy(xp0_hbm.at[iv.at[0]], rows_v, gsem).wait()
        pltpu.make_async_copy(al_hbm.at[iv.at[1]], ald_v, gsem).wait()

    def scat(slot, rows_v, ssem):
        pltpu.async_copy(rows_v, acc_sp.at[idxs[slot].at[1]], ssem, add=True)

    def scat_wait(slot, rows_v, ssem):
        pltpu.make_async_copy(rows_v, acc_sp.at[idxs[slot].at[1]],
                              ssem).wait()

    def compute_chunk(ald_v, rows_v):
        for g in range(EK // LANES):
            erow = lane_iota + g * LANES
            for h in range(2):
                a_s = plsc.load_gather(
                    rows_v, [erow, jnp.full((LANES,), D + 2 + h, jnp.int32)])
                a_d = plsc.load_gather(
                    ald_v,
                    [erow, jnp.full((LANES,), H + h, jnp.int32) + hh0])
                al = a_s + a_d
                alpha = jnp.where(al >= 0.0, al, 0.2 * al)
                ex = jnp.exp(alpha)
                plsc.store_scatter(
                    rows_v, [erow, jnp.full((LANES,), D + h, jnp.int32)], ex)

        for e in range(EK):
            exv = rows_v[e, pl.ds(D, LANES)]
            b0 = jnp.full((LANES,), exv[0], jnp.float32)
            b1 = jnp.full((LANES,), exv[1], jnp.float32)
            for q in range(C // LANES):
                rows_v[e, pl.ds(q * LANES, LANES)] = (
                    rows_v[e, pl.ds(q * LANES, LANES)] * b0)
            for q in range(C // LANES, 2 * C // LANES):
                rows_v[e, pl.ds(q * LANES, LANES)] = (
                    rows_v[e, pl.ds(q * LANES, LANES)] * b1)

    idx_load(0, 0)
    idx_load(1, 1)
    idx_load(2, 2)
    idx_load(3, 3)
    idx_wait(0)
    start_gathers(0, rbufs[0][0], rbufs[0][1], rbufs[0][2])

    def body(j, carry):
        for b in range(6):
            k = 6 * j + b
            A = rbufs[b % 3]
            B = rbufs[(b + 1) % 3]
            sl = b
            sln = (b + 1) % 6
            slf = (b + 4) % 6

            def drain_prev():
                scat_wait(slf, B[1], B[3])

            if b < 2:
                @pl.when(j > 0)
                def _():
                    drain_prev()
            else:
                drain_prev()

            if b < 2:
                idx_load(k + 4, slf)
            else:
                @pl.when(j < JN - 1)
                def _():
                    idx_load(k + 4, slf)

            def fetch_next():
                idx_wait(sln)
                start_gathers(sln, B[0], B[1], B[2])

            if b < 5:
                fetch_next()
            else:
                @pl.when(j < JN - 1)
                def _():
                    fetch_next()

            wait_gathers(sl, A[0], A[1], A[2])
            compute_chunk(A[0], A[1])
            scat(sl, A[1], A[3])
        return carry

    lax.fori_loop(0, JN, body, 0)

    scat_wait(4, rbufs[1][1], rbufs[1][3])
    scat_wait(5, rbufs[2][1], rbufs[2][3])
    plsc.subcore_barrier()

    pltpu.sync_copy(acc_sp.at[pl.ds(r0, RPT)],
                    out_hbm.at[c, pl.ds(r0, RPT)])


@functools.lru_cache(maxsize=1)
def _sc_gat_call():
    return pl.kernel(
        _sc_gat,
        out_type=jax.ShapeDtypeStruct((NC, NPAD, W_ROW), jnp.float32),
        mesh=plsc.VectorSubcoreMesh(core_axis_name="c", subcore_axis_name="s",
                                    num_cores=NC, num_subcores=NS),
        compiler_params=pltpu.CompilerParams(needs_layout_passes=False,
                                             use_tc_tiling_on_sc=False),
        scratch_types=[
            pltpu.VMEM((2, EK), jnp.int32),
            pltpu.VMEM((2, EK), jnp.int32),
            pltpu.VMEM((2, EK), jnp.int32),
            pltpu.VMEM((2, EK), jnp.int32),
            pltpu.VMEM((2, EK), jnp.int32),
            pltpu.VMEM((2, EK), jnp.int32),
            pltpu.VMEM((EK, 16), jnp.float32),
            pltpu.VMEM((EK, W_ROW), jnp.float32),
            pltpu.VMEM((EK, 16), jnp.float32),
            pltpu.VMEM((EK, W_ROW), jnp.float32),
            pltpu.VMEM((EK, 16), jnp.float32),
            pltpu.VMEM((EK, W_ROW), jnp.float32),
            pltpu.VMEM_SHARED((NPAD, W_ROW), jnp.float32),
            pltpu.SemaphoreType.DMA,
            pltpu.SemaphoreType.DMA,
            pltpu.SemaphoreType.DMA,
            pltpu.SemaphoreType.DMA,
            pltpu.SemaphoreType.DMA,
            pltpu.SemaphoreType.DMA,
            pltpu.SemaphoreType.DMA,
            pltpu.SemaphoreType.DMA,
            pltpu.SemaphoreType.DMA,
            pltpu.SemaphoreType.DMA,
            pltpu.SemaphoreType.DMA,
            pltpu.SemaphoreType.DMA,
        ],
    )


def _tc_post(acc0_ref, acc1_ref, b0_ref, b1_ref,
             saw1_ref, sab1_ref, saw2_ref, z0_ref, z1_ref, wsum_ref):
    i = pl.program_id(0)
    saw1 = saw1_ref[...]
    sab1 = sab1_ref[...]
    saw2 = saw2_ref[...]
    wparts = []
    for acc_ref, b_ref, z_ref in (
            (acc0_ref, b0_ref, z0_ref),
            (acc1_ref, b1_ref, z1_ref)):
        acc = acc_ref[...]
        segs = []
        for sc in range(2):
            for h in range(2):
                num = acc[sc, :, h * C:(h + 1) * C]
                den = acc[sc, :, D + h][:, None]
                segs.append(num / (den + 1e-16))
        z = jnp.concatenate(segs, axis=1) + b_ref[...]
        z_ref[...] = z
        t = jnp.tanh(
            lax.dot_general(z, saw1, (((1,), (0,)), ((), ())),
                            precision=lax.Precision.HIGHEST,
                            preferred_element_type=jnp.float32) + sab1)
        wparts.append(jnp.sum(t * saw2))
    wnew = jnp.stack([jnp.full((128,), wparts[0], jnp.float32),
                      jnp.full((128,), wparts[1], jnp.float32)])

    @pl.when(i == 0)
    def _():
        wsum_ref[...] = wnew

    @pl.when(i > 0)
    def _():
        wsum_ref[...] = wsum_ref[...] + wnew


_tc_post_call = pl.pallas_call(
    _tc_post,
    grid=(NBLK,),
    in_specs=[
        pl.BlockSpec((NC, ROWBLK, W_ROW), lambda i: (0, i, 0)),
        pl.BlockSpec((NC, ROWBLK, W_ROW), lambda i: (0, i, 0)),
        pl.BlockSpec((1, HC), lambda i: (0, 0)),
        pl.BlockSpec((1, HC), lambda i: (0, 0)),
        pl.BlockSpec((HC, HID), lambda i: (0, 0)),
        pl.BlockSpec((1, HID), lambda i: (0, 0)),
        pl.BlockSpec((1, HID), lambda i: (0, 0)),
    ],
    out_specs=[
        pl.BlockSpec((ROWBLK, HC), lambda i: (i, 0)),
        pl.BlockSpec((ROWBLK, HC), lambda i: (i, 0)),
        pl.BlockSpec((2, 128), lambda i: (0, 0)),
    ],
    out_shape=(
        jax.ShapeDtypeStruct((N, HC), jnp.float32),
        jax.ShapeDtypeStruct((N, HC), jnp.float32),
        jax.ShapeDtypeStruct((2, 128), jnp.float32),
    ),
)


def _tc_combine(z0_ref, z1_ref, wsum_ref, out_ref):
    w = wsum_ref[...]
    a0 = w[0:1, 0:1] / N
    a1 = w[1:2, 0:1] / N
    m = jnp.maximum(a0, a1)
    e0 = jnp.exp(a0 - m)
    e1 = jnp.exp(a1 - m)
    tot = e0 + e1
    out_ref[...] = (z0_ref[...] * (e0 / tot) + z1_ref[...] * (e1 / tot))


_tc_combine_call = pl.pallas_call(
    _tc_combine,
    grid=(NBLK,),
    in_specs=[
        pl.BlockSpec((ROWBLK, HC), lambda i: (i, 0)),
        pl.BlockSpec((ROWBLK, HC), lambda i: (i, 0)),
        pl.BlockSpec((2, 128), lambda i: (0, 0)),
    ],
    out_specs=pl.BlockSpec((ROWBLK, HC), lambda i: (i, 0)),
    out_shape=jax.ShapeDtypeStruct((N, HC), jnp.float32),
)


def kernel(h, edge_index_0, edge_index_1, W0, as0, ad0, b0,
           W1, as1, ad1, b1, saW1, sab1, saW2):
    rows = jnp.arange(HC)
    headid = (rows // C).astype(jnp.int32)
    asd0 = jnp.zeros((HC, 16), jnp.float32)
    asd0 = asd0.at[rows, headid].set(as0.reshape(HC))
    asd0 = asd0.at[rows, H + headid].set(ad0.reshape(HC))
    asd1 = jnp.zeros((HC, 16), jnp.float32)
    asd1 = asd1.at[rows, headid].set(as1.reshape(HC))
    asd1 = asd1.at[rows, H + headid].set(ad1.reshape(HC))

    loop = jnp.arange(N, dtype=jnp.int32)
    pad = jnp.full((E2 - E - N,), PAD_NODE, jnp.int32)

    def pack(ei):
        src = jnp.concatenate([ei[0].astype(jnp.int32), loop, pad])
        dst = jnp.concatenate([ei[1].astype(jnp.int32), loop, pad])
        return jnp.stack([src.reshape(-1, EK), dst.reshape(-1, EK)], axis=1)

    idx0 = pack(edge_index_0)
    idx1 = pack(edge_index_1)

    h_pad = jnp.zeros((NPAD, D), jnp.float32).at[:N].set(h)
    xp00, xp01, xp10, xp11, alp0, alp1 = _tc_prep_call(
        h_pad, W0, W1, asd0, asd1)

    sc_gat = _sc_gat_call()
    acc0 = sc_gat(idx0, alp0, xp00, xp01)
    acc1 = sc_gat(idx1, alp1, xp10, xp11)

    z0, z1, wsum = _tc_post_call(
        acc0, acc1, b0.reshape(1, HC), b1.reshape(1, HC),
        saW1, sab1.reshape(1, HID), saW2.reshape(1, HID))

    return _tc_combine_call(z0, z1, wsum)
